# baseline (speedup 1.0000x reference)
import functools

import jax
import jax.numpy as jnp
from jax import lax
from jax.experimental import pallas as pl
from jax.experimental.pallas import tpu as pltpu
from jax.experimental.pallas import tpu_sc as plsc

NC = 2
NS = 16
NW = NC * NS

BATCH = 128
GROUP = 8


@functools.lru_cache(maxsize=None)
def _make_gather(n_rows: int, dim: int, dim_pad: int):
    assert n_rows % (NW * GROUP) == 0
    rows_per_w = n_rows // NW
    n_groups = rows_per_w // GROUP
    g_words = GROUP * BATCH * dim

    n_vec = dim // 16
    tail = dim - 16 * n_vec

    mesh = plsc.VectorSubcoreMesh(core_axis_name="c", subcore_axis_name="s")

    @functools.partial(
        pl.kernel,
        mesh=mesh,
        out_type=jax.ShapeDtypeStruct((n_rows * BATCH * dim,), jnp.float32),
        scratch_types=[
            pltpu.VMEM((GROUP, BATCH), jnp.int32),
            pltpu.VMEM((GROUP, BATCH, dim_pad), jnp.float32),
            pltpu.VMEM((g_words + 16,), jnp.float32),
            pltpu.SemaphoreType.DMA,
        ],
        compiler_params=pltpu.CompilerParams(use_tc_tiling_on_sc=False),
    )
    def k(emb_hbm, xg_hbm, out_hbm, idx_v, rows_v, comp_v, sem):
        wid = lax.axis_index("s") * NC + lax.axis_index("c")
        row_base = wid * rows_per_w

        def group_body(g, carry):
            gbase = row_base + g * GROUP
            pltpu.sync_copy(xg_hbm.at[pl.ds(gbase, GROUP)], idx_v)
            handles = [
                pltpu.async_copy(emb_hbm.at[idx_v.at[j]], rows_v.at[j], sem)
                for j in range(GROUP)
            ]
            for h in handles:
                h.wait()

            for j in range(GROUP):

                def row_body(r, off, j=j):
                    for v in range(n_vec):
                        comp_v[pl.ds(off + 16 * v, 16)] = rows_v[
                            j, r, pl.ds(16 * v, 16)
                        ]
                    if tail:
                        comp_v[pl.ds(off + 16 * n_vec, 16)] = rows_v[
                            j, r, pl.ds(16 * n_vec, 16)
                        ]
                    return off + dim

                lax.fori_loop(0, BATCH, row_body, j * BATCH * dim)
            pltpu.sync_copy(
                comp_v.at[pl.ds(0, g_words)],
                out_hbm.at[pl.ds(gbase * BATCH * dim, g_words)],
            )
            return carry

        lax.fori_loop(0, n_groups, group_body, 0)

    return k


def kernel(x, emb):
    b, s = x.shape
    v, d = emb.shape
    n = b * s
    assert n % BATCH == 0
    d_pad = (d + 15) // 16 * 16
    emb_p = jnp.pad(emb, ((0, 0), (0, d_pad - d))) if d_pad != d else emb
    xg = x.reshape(n // BATCH, BATCH).astype(jnp.int32)
    out = _make_gather(n // BATCH, d, d_pad)(emb_p.astype(jnp.float32), xg)
    return out.reshape(b, s, d)


# speedup vs baseline: 1.3056x; 1.3056x over previous
import functools

import jax
import jax.numpy as jnp
from jax import lax
from jax.experimental import pallas as pl
from jax.experimental.pallas import tpu as pltpu
from jax.experimental.pallas import tpu_sc as plsc

NC = 2
NS = 16
NW = NC * NS

BATCH = 128
GROUP = 4
UNROLL = 4


@functools.lru_cache(maxsize=None)
def _make_gather(n_rows: int, dim: int, dim_pad: int):
    assert n_rows % (NW * GROUP) == 0
    rows_per_w = n_rows // NW
    n_groups = rows_per_w // GROUP
    assert n_groups % 2 == 0
    g_words = GROUP * BATCH * dim

    n_vec = dim // 16
    tail = dim - 16 * n_vec

    mesh = plsc.VectorSubcoreMesh(core_axis_name="c", subcore_axis_name="s")

    @functools.partial(
        pl.kernel,
        mesh=mesh,
        out_type=jax.ShapeDtypeStruct((n_rows * BATCH * dim,), jnp.float32),
        scratch_types=[
            pltpu.VMEM((2, GROUP, BATCH), jnp.int32),
            pltpu.VMEM((2, GROUP, BATCH, dim_pad), jnp.float32),
            pltpu.VMEM((g_words + 16,), jnp.float32),
            pltpu.SemaphoreType.DMA,
        ],
        compiler_params=pltpu.CompilerParams(use_tc_tiling_on_sc=False),
    )
    def k(emb_hbm, xg_hbm, out_hbm, idx_v, rows_v, comp_v, sem_g):
        wid = lax.axis_index("s") * NC + lax.axis_index("c")
        row_base = wid * rows_per_w

        def fire(buf, g):
            gbase = row_base + g * GROUP
            pltpu.sync_copy(xg_hbm.at[pl.ds(gbase, GROUP)], idx_v.at[buf])
            for j in range(GROUP):
                pltpu.async_copy(
                    emb_hbm.at[idx_v.at[buf, j]], rows_v.at[buf, j], sem_g
                )

        def drain(buf):
            for j in range(GROUP):
                pltpu.make_async_copy(
                    emb_hbm.at[idx_v.at[buf, j]], rows_v.at[buf, j], sem_g
                ).wait()

        def process(buf, g):
            for j in range(GROUP):

                def quad_body(rr, off, j=j):
                    r0 = rr * UNROLL
                    vecs = [
                        rows_v[buf, j, r0 + q, pl.ds(16 * v, 16)]
                        for q in range(UNROLL)
                        for v in range(n_vec + (1 if tail else 0))
                    ]
                    i = 0
                    for q in range(UNROLL):
                        o = off + q * dim
                        for v in range(n_vec):
                            comp_v[pl.ds(o + 16 * v, 16)] = vecs[i]
                            i += 1
                        if tail:
                            comp_v[pl.ds(o + 16 * n_vec, 16)] = vecs[i]
                            i += 1
                    return off + UNROLL * dim

                lax.fori_loop(0, BATCH // UNROLL, quad_body, j * BATCH * dim)

            pltpu.sync_copy(
                comp_v.at[pl.ds(0, g_words)],
                out_hbm.at[pl.ds((row_base + g * GROUP) * BATCH * dim,
                                 g_words)],
            )

        fire(0, 0)

        def pair_body(gg, carry):
            for b in range(2):
                g = gg * 2 + b
                drain(b)

                @pl.when(g + 1 < n_groups)
                def _():
                    fire(1 - b, g + 1)

                process(b, g)
            return carry

        lax.fori_loop(0, n_groups // 2, pair_body, 0)

    return k


def kernel(x, emb):
    b, s = x.shape
    v, d = emb.shape
    n = b * s
    assert n % BATCH == 0
    d_pad = (d + 15) // 16 * 16
    emb_p = jnp.pad(emb, ((0, 0), (0, d_pad - d))) if d_pad != d else emb
    xg = x.reshape(n // BATCH, BATCH).astype(jnp.int32)
    out = _make_gather(n // BATCH, d, d_pad)(emb_p.astype(jnp.float32), xg)
    return out.reshape(b, s, d)


# speedup vs baseline: 2.6267x; 2.0118x over previous
import functools

import jax
import jax.numpy as jnp
from jax import lax
from jax.experimental import pallas as pl
from jax.experimental.pallas import tpu as pltpu
from jax.experimental.pallas import tpu_sc as plsc

NC = 2
NS = 16
NW = NC * NS

BATCH = 128
SUPER = 8
GROUP = 2


@functools.lru_cache(maxsize=None)
def _make_gather(n_rows: int, dim: int, lanes: int):
    assert n_rows % (NW * SUPER) == 0
    rows_per_w = n_rows // NW
    n_super = rows_per_w // SUPER

    n_vec = dim // 16
    tail = dim - 16 * n_vec
    t_off = dim - 16

    mesh = plsc.VectorSubcoreMesh(core_axis_name="c", subcore_axis_name="s")

    @functools.partial(
        pl.kernel,
        mesh=mesh,
        out_type=jax.ShapeDtypeStruct((n_rows * BATCH, dim), jnp.float32),
        scratch_types=[
            pltpu.VMEM((2, SUPER, BATCH), jnp.int32),
            pltpu.VMEM((2, GROUP, BATCH, lanes), jnp.float32),
            pltpu.VMEM((GROUP * BATCH, dim), jnp.float32),
            pltpu.SemaphoreType.DMA,
        ],
        compiler_params=pltpu.CompilerParams(use_tc_tiling_on_sc=True),
    )
    def k(emb_hbm, xg_hbm, out_hbm, idx_v, rows_v, comp_v, sem_g):
        wid = lax.axis_index("s") * NC + lax.axis_index("c")
        row_base = wid * rows_per_w

        def fetch_idx(sb, sg):
            pltpu.sync_copy(
                xg_hbm.at[pl.ds(row_base + sg * SUPER, SUPER)], idx_v.at[sb]
            )

        def fire(buf, sb, jj):
            for j in range(GROUP):
                pltpu.async_copy(
                    emb_hbm.at[idx_v.at[sb, jj + j]], rows_v.at[buf, j], sem_g
                )

        def drain(buf, sb, jj):
            for j in range(GROUP):
                pltpu.make_async_copy(
                    emb_hbm.at[idx_v.at[sb, jj + j]], rows_v.at[buf, j], sem_g
                ).wait()

        def process(buf, g):
            for j in range(GROUP):

                def row_body(r, carry, j=j):
                    d = j * BATCH + r
                    for v in range(n_vec):
                        comp_v[d, pl.ds(16 * v, 16)] = rows_v[
                            buf, j, r, pl.ds(16 * v, 16)
                        ]
                    if tail:
                        comp_v[d, pl.ds(t_off, 16)] = rows_v[
                            buf, j, r, pl.ds(t_off, 16)
                        ]
                    return carry

                lax.fori_loop(0, BATCH, row_body, 0)

            pltpu.sync_copy(
                comp_v,
                out_hbm.at[pl.ds((row_base + g * GROUP) * BATCH,
                                 GROUP * BATCH)],
            )

        n_grp = SUPER // GROUP

        fetch_idx(0, 0)
        fire(0, 0, 0)

        def super_body(sg, carry):
            sb = lax.rem(sg, 2)

            @pl.when(sg + 1 < n_super)
            def _():
                fetch_idx(1 - sb, sg + 1)

            for i in range(n_grp):
                b = i % 2
                drain(b, sb, i * GROUP)
                if i + 1 < n_grp:
                    fire(1 - b, sb, (i + 1) * GROUP)
                else:

                    @pl.when(sg + 1 < n_super)
                    def _():
                        fire(1 - b, 1 - sb, 0)

                process(b, sg * n_grp + i)
            return carry

        lax.fori_loop(0, n_super, super_body, 0)

    return k


def kernel(x, emb):
    b, s = x.shape
    v, d = emb.shape
    n = b * s
    assert n % BATCH == 0
    lanes = 128
    emb_p = jnp.pad(emb, ((0, 0), (0, lanes - d)))
    xg = x.reshape(n // BATCH, BATCH).astype(jnp.int32)
    out = _make_gather(n // BATCH, d, lanes)(emb_p.astype(jnp.float32), xg)
    return out.reshape(b, s, d)
